# Initial kernel scaffold; baseline (speedup 1.0000x reference)
import jax
import jax.numpy as jnp
from jax.experimental import pallas as pl


def kernel(data, point_to_pixel_feat, emb, params):
    raise NotImplementedError("write your pallas kernel here")


# speedup vs baseline: 5.4377x; 5.4377x over previous
import functools

import jax
import jax.numpy as jnp
from jax import lax
from jax.experimental import pallas as pl
from jax.experimental.pallas import tpu as pltpu
from jax.experimental.pallas import tpu_sc as plsc

_EPS = 1e-5


def _dot(a, b):
    return jnp.dot(a.astype(jnp.bfloat16), b.astype(jnp.bfloat16),
                   preferred_element_type=jnp.float32)


def _dotg(a, b, dn):
    return lax.dot_general(a.astype(jnp.bfloat16), b.astype(jnp.bfloat16), dn,
                           preferred_element_type=jnp.float32)


def _sc_gather(table, idx):
    V, C = table.shape
    if C % 128 != 0:
        Cp = ((C + 127) // 128) * 128
        table = jnp.concatenate(
            [table, jnp.zeros((V, Cp - C), jnp.float32)], axis=1)
        return _sc_gather(table, idx)[:, :C]
    (M,) = idx.shape
    NW = 32
    assert M % NW == 0, M
    bpw = M // NW
    nch = max(1, bpw // 128)
    ch = bpw // nch
    assert ch * nch == bpw and ch % 8 == 0 and ch <= 128, (bpw, nch, ch)
    mesh = plsc.VectorSubcoreMesh(core_axis_name="c", subcore_axis_name="s")

    @functools.partial(
        pl.kernel,
        mesh=mesh,
        out_type=jax.ShapeDtypeStruct((M, C), jnp.float32),
        scratch_types=[
            pltpu.VMEM((bpw,), jnp.int32),
            pltpu.VMEM((ch, C), jnp.float32),
            pltpu.SemaphoreType.DMA,
        ],
    )
    def k(table_hbm, idx_hbm, out_hbm, idx_v, rows_v, sem):
        wid = lax.axis_index("s") * 2 + lax.axis_index("c")
        base = wid * bpw
        pltpu.sync_copy(idx_hbm.at[pl.ds(base, bpw)], idx_v)
        for c0 in range(nch):
            pltpu.async_copy(
                table_hbm.at[idx_v.at[pl.ds(c0 * ch, ch)]], rows_v, sem
            ).wait()
            pltpu.sync_copy(rows_v, out_hbm.at[pl.ds(base + c0 * ch, ch)])

    return k(table, idx)


def _bn_tail(y, nrows, g, b, relu):
    mu = jnp.sum(y, axis=0, keepdims=True) * (1.0 / nrows)
    d = y - mu
    var = jnp.sum(d * d, axis=0, keepdims=True) * (1.0 / nrows)
    out = d / jnp.sqrt(var + _EPS) * g + b
    if relu:
        out = jnp.maximum(out, 0.0)
    return out


def _mm_bn_relu_kernel(x_ref, w_ref, bias_ref, g_ref, b_ref, o_ref, *, nrows, relu):
    y = _dot(x_ref[...], w_ref[...])
    y = y + bias_ref[...]
    o_ref[...] = _bn_tail(y, nrows, g_ref[...], b_ref[...], relu)


def _linear_bn_relu(x, W, bias, g, b, relu=True):
    R, _ = x.shape
    C = W.shape[1]
    return pl.pallas_call(
        functools.partial(_mm_bn_relu_kernel, nrows=R, relu=relu),
        out_shape=jax.ShapeDtypeStruct((R, C), jnp.float32),
    )(x, W, bias.reshape(1, C), g.reshape(1, C), b.reshape(1, C))


def _head_kernel(x_ref, w1_ref, b1_ref, g_ref, bb_ref, w2_ref, b2_ref, o_ref, *, nrows):
    y = _dot(x_ref[...], w1_ref[...])
    y = y + b1_ref[...]
    h = _bn_tail(y, nrows, g_ref[...], bb_ref[...], True)
    o_ref[...] = _dot(h, w2_ref[...]) + b2_ref[...]


def _head(x, W1, b1, g, bb, W2, b2):
    R = x.shape[0]
    C2 = W2.shape[1]
    return pl.pallas_call(
        functools.partial(_head_kernel, nrows=R),
        out_shape=jax.ShapeDtypeStruct((R, C2), jnp.float32),
    )(x, W1, b1.reshape(1, -1), g.reshape(1, -1), bb.reshape(1, -1), W2,
      b2.reshape(1, C2))


def _pe_kernel(pix_ref, emb_ref, o_ref, *, nrows):
    oh = (pix_ref[...] == lax.broadcasted_iota(jnp.int32, (nrows, 32), 1)
          ).astype(jnp.float32)
    o_ref[...] = jnp.dot(oh, emb_ref[...], preferred_element_type=jnp.float32,
                         precision=lax.Precision.HIGHEST)


def _pe(pix, emb):
    R = pix.shape[0]
    emb32 = jnp.concatenate(
        [emb, jnp.zeros((32 - emb.shape[0], emb.shape[1]), jnp.float32)], axis=0)
    return pl.pallas_call(
        functools.partial(_pe_kernel, nrows=R),
        out_shape=jax.ShapeDtypeStruct((R, emb.shape[1]), jnp.float32),
    )(pix, emb32)


def _fps_kernel(pt_ref, sel_ref, *, n, m):
    pt = pt_ref[0]
    lane_n = lax.broadcasted_iota(jnp.int32, (1, n), 1)
    col_m8 = lax.broadcasted_iota(jnp.int32, (8, m), 1)
    c0 = pt[:, 0:1]
    sel0 = jnp.where(col_m8 == 0, c0, 0.0)
    dists0 = jnp.full((1, n), 1e10, jnp.float32)

    def body(i, st):
        dists, sel, c = st
        diff = pt - c
        d2 = diff * diff
        dcur = (d2[0:1] + d2[1:2]) + d2[2:3]
        dists = jnp.minimum(dists, dcur)
        mx = jnp.max(dists)
        nxt = jnp.min(jnp.where(dists == mx, lane_n, n))
        cnxt = jnp.sum(jnp.where(lane_n == nxt, pt, 0.0), axis=1, keepdims=True)
        sel = jnp.where(col_m8 == i, cnxt, sel)
        return dists, sel, cnxt

    _, sel, _ = lax.fori_loop(1, m, body, (dists0, sel0, c0))
    sel_ref[0] = sel


def _fps(pt, m):
    B, _, n = pt.shape
    return pl.pallas_call(
        functools.partial(_fps_kernel, n=n, m=m),
        grid=(B,),
        in_specs=[pl.BlockSpec((1, 8, n), lambda b: (b, 0, 0))],
        out_specs=pl.BlockSpec((1, 8, m), lambda b: (b, 0, 0)),
        out_shape=jax.ShapeDtypeStruct((B, 8, m), jnp.float32),
    )(pt)


def _dist_block(qt, pt):
    qd = _dotg(qt, pt, (((0,), (0,)), ((), ())))
    ones = jnp.ones((8, 1), jnp.float32)
    qn = lax.dot_general(qt * qt, ones, (((0,), (0,)), ((), ())),
                         preferred_element_type=jnp.float32,
                         precision=lax.Precision.HIGHEST)
    p2 = pt * pt
    rn = (p2[0:1] + p2[1:2]) + p2[2:3]
    return qn + rn - 2.0 * qd


def _knn_kernel(qt_ref, pt_ref, idx_ref, wt_ref, *, n, k, blk, n_full, with_w):
    b = pl.program_id(0)
    d = _dist_block(qt_ref[0], pt_ref[0])
    lane_n = lax.broadcasted_iota(jnp.int32, (blk, n), 1)
    col_k = lax.broadcasted_iota(jnp.int32, (blk, k), 1)
    idxs = jnp.zeros((blk, k), jnp.int32)
    vals = jnp.zeros((blk, k), jnp.float32)
    for j in range(k):
        mn = jnp.min(d, axis=1, keepdims=True)
        sel = jnp.min(jnp.where(d == mn, lane_n, n), axis=1, keepdims=True)
        d = jnp.where(lane_n == sel, 1e30, d)
        idxs = jnp.where(col_k == j, sel, idxs)
        if with_w:
            vals = jnp.where(col_k == j, mn, vals)
    idx_ref[0] = idxs + b * n_full
    if with_w:
        sq = jnp.maximum(vals, 0.0)
        w = 1.0 / (sq + 1e-8)
        wt_ref[0] = w / jnp.sum(w, axis=1, keepdims=True)


def _knn(qt, pt, k, with_w):
    B, _, m = qt.shape
    n = pt.shape[2]
    blk = min(m, 256)
    grid = (B, m // blk)
    out_shapes = [jax.ShapeDtypeStruct((B, m, k), jnp.int32),
                  jax.ShapeDtypeStruct((B, m, k), jnp.float32)]
    idx, wt = pl.pallas_call(
        functools.partial(_knn_kernel, n=n, k=k, blk=blk, n_full=n, with_w=with_w),
        grid=grid,
        in_specs=[pl.BlockSpec((1, 8, blk), lambda b, j: (b, 0, j)),
                  pl.BlockSpec((1, 8, n), lambda b, j: (b, 0, 0))],
        out_specs=[pl.BlockSpec((1, blk, k), lambda b, j: (b, j, 0)),
                   pl.BlockSpec((1, blk, k), lambda b, j: (b, j, 0))],
        out_shape=out_shapes,
    )(qt, pt)
    return idx, wt


def _td_y(g_ref, npx_ref, w_ref, blk, k, C8):
    d = g_ref[...] - npx_ref[...][:, None, :]
    return _dot(d.reshape(blk * k, C8), w_ref[...])


def _td_stats_kernel(g_ref, npx_ref, w_ref, s1_ref, s2_ref, *, blk, k, C8, C):
    y = _td_y(g_ref, npx_ref, w_ref, blk, k, C8)
    s1_ref[...] = jnp.sum(y, axis=0).reshape(1, 1, C)
    s2_ref[...] = jnp.sum(y * y, axis=0).reshape(1, 1, C)


def _td_final_kernel(s1_ref, s2_ref, g_ref, npx_ref, w_ref, gam_ref, bet_ref,
                     o_ref, *, nblk, blk, k, C8, C, ntot):
    s1 = jnp.sum(s1_ref[...].reshape(nblk, C), axis=0, keepdims=True)
    s2 = jnp.sum(s2_ref[...].reshape(nblk, C), axis=0, keepdims=True)
    mu = s1 * (1.0 / ntot)
    var = s2 * (1.0 / ntot) - mu * mu
    scale = gam_ref[...] / jnp.sqrt(var + _EPS)
    y = _td_y(g_ref, npx_ref, w_ref, blk, k, C8)
    yn = (y - mu) * scale + bet_ref[...]
    o_ref[...] = jnp.max(jnp.maximum(yn, 0.0).reshape(blk, k, C), axis=1)


def _td_bn_pool(g3, npx, Wf, gam, bet):
    R, k, C8 = g3.shape
    C = Wf.shape[1]
    blk = min(R, 256)
    nblk = R // blk
    wspec = pl.BlockSpec((C8, C), lambda i: (0, 0))
    s1, s2 = pl.pallas_call(
        functools.partial(_td_stats_kernel, blk=blk, k=k, C8=C8, C=C),
        grid=(nblk,),
        in_specs=[pl.BlockSpec((blk, k, C8), lambda i: (i, 0, 0)),
                  pl.BlockSpec((blk, C8), lambda i: (i, 0)),
                  wspec],
        out_specs=[pl.BlockSpec((1, 1, C), lambda i: (i, 0, 0)),
                   pl.BlockSpec((1, 1, C), lambda i: (i, 0, 0))],
        out_shape=[jax.ShapeDtypeStruct((nblk, 1, C), jnp.float32),
                   jax.ShapeDtypeStruct((nblk, 1, C), jnp.float32)],
    )(g3, npx, Wf)
    out = pl.pallas_call(
        functools.partial(_td_final_kernel, nblk=nblk, blk=blk, k=k, C8=C8,
                          C=C, ntot=R * k),
        grid=(nblk,),
        in_specs=[pl.BlockSpec((nblk, 1, C), lambda i: (0, 0, 0)),
                  pl.BlockSpec((nblk, 1, C), lambda i: (0, 0, 0)),
                  pl.BlockSpec((blk, k, C8), lambda i: (i, 0, 0)),
                  pl.BlockSpec((blk, C8), lambda i: (i, 0)),
                  wspec,
                  pl.BlockSpec((1, C), lambda i: (0, 0)),
                  pl.BlockSpec((1, C), lambda i: (0, 0))],
        out_specs=pl.BlockSpec((blk, C), lambda i: (i, 0)),
        out_shape=jax.ShapeDtypeStruct((R, C), jnp.float32),
    )(s1, s2, g3, npx, Wf, gam.reshape(1, C), bet.reshape(1, C))
    return out


def _dec5_kernel(x5_ref, w2_ref, b2_ref, w1a_ref, w1b_ref, b1_ref, g_ref,
                 bb_ref, o_ref, *, B, npt, C):
    x5 = x5_ref[...]
    avg = jnp.sum(x5, axis=1) * (1.0 / npt)
    gf = jnp.maximum(_dot(avg, w2_ref[...]) + b2_ref[...], 0.0)
    y = (_dot(x5.reshape(B * npt, C), w1a_ref[...]).reshape(B, npt, C)
         + _dot(gf, w1b_ref[...])[:, None, :]
         + b1_ref[...][None])
    o_ref[...] = _bn_tail(y.reshape(B * npt, C), B * npt, g_ref[...],
                          bb_ref[...], True)


def _dec5(x5_flat, B, npt, P):
    C = x5_flat.shape[1]
    W1 = P["dec5_l1_W"]
    return pl.pallas_call(
        functools.partial(_dec5_kernel, B=B, npt=npt, C=C),
        out_shape=jax.ShapeDtypeStruct((B * npt, C), jnp.float32),
    )(x5_flat.reshape(B, npt, C), P["dec5_l2_W"],
      P["dec5_l2_b"].reshape(1, C), W1[:C], W1[C:],
      P["dec5_l1_b"].reshape(1, C), P["dec5_l1_g"].reshape(1, C),
      P["dec5_l1_bb"].reshape(1, C))


def _interp_combine_kernel(a_ref, g0_ref, g1_ref, g2_ref, w0_ref, w1_ref,
                           w2_ref, o_ref):
    o_ref[...] = (a_ref[...]
                  + g0_ref[...] * w0_ref[...]
                  + g1_ref[...] * w1_ref[...]
                  + g2_ref[...] * w2_ref[...])


def _interp_combine(a, g0, g1, g2, w0, w1, w2):
    R, C = a.shape
    blk = min(R, 2048)
    rb = pl.BlockSpec((blk, C), lambda i: (i, 0))
    wb = pl.BlockSpec((blk, 1), lambda i: (i, 0))
    return pl.pallas_call(
        _interp_combine_kernel,
        grid=(R // blk,),
        in_specs=[rb, rb, rb, rb, wb, wb, wb],
        out_specs=rb,
        out_shape=jax.ShapeDtypeStruct((R, C), jnp.float32),
    )(a, g0, g1, g2, w0, w1, w2)


def _pad_pt(sel):
    return sel


def kernel(data, point_to_pixel_feat, emb, params):
    P = params
    B, _, N = data.shape
    f32 = jnp.float32

    dataT = jnp.transpose(data, (0, 2, 1)).reshape(B * N, 6)
    pix = jnp.transpose(point_to_pixel_feat, (0, 2, 1))[:, :, 0]
    pix = pix.reshape(B * N, 1).astype(jnp.int32)

    pe = _pe(pix, emb)
    x10 = jnp.concatenate([dataT, pe], axis=1)
    x1 = _linear_bn_relu(x10, P["enc1_W"],
                         jnp.zeros((P["enc1_W"].shape[1],), f32),
                         P["enc1_g"], P["enc1_b"])

    p1t = jnp.concatenate(
        [data[:, :3, :], jnp.zeros((B, 5, N), f32)], axis=1)

    def td(pt, x_flat, lvl):
        n = pt.shape[2]
        m = n // 4
        W = P["enc%d_W" % lvl]
        Ci = W.shape[0] - 3
        Co = W.shape[1]
        C8 = Ci + 8
        Wf = jnp.concatenate([W[:3], jnp.zeros((5, Co), f32), W[3:]], axis=0)
        selt = _fps(pt, m)
        nbr, _ = _knn(selt, pt, 16, False)
        tbl = jnp.concatenate(
            [jnp.transpose(pt, (0, 2, 1)),
             x_flat.reshape(B, n, Ci)], axis=2).reshape(B * n, C8)
        npx = jnp.concatenate(
            [jnp.transpose(selt, (0, 2, 1)),
             jnp.zeros((B, m, Ci), f32)], axis=2).reshape(B * m, C8)
        g = _sc_gather(tbl, nbr.reshape(-1))
        x_out = _td_bn_pool(g.reshape(B * m, 16, C8), npx, Wf,
                            P["enc%d_g" % lvl], P["enc%d_b" % lvl])
        return selt, x_out

    p2t, x2 = td(p1t, x1, 2)
    p3t, x3 = td(p2t, x2, 3)
    p4t, x4 = td(p3t, x3, 4)
    p5t, x5 = td(p4t, x4, 5)

    x5d = _dec5(x5, B, p5t.shape[2], P)

    def up(xf, ptf, xc, ptc, nm):
        nf, nc = ptf.shape[2], ptc.shape[2]
        a = _linear_bn_relu(xf, P[nm + "_l1_W"], P[nm + "_l1_b"],
                            P[nm + "_l1_g"], P[nm + "_l1_bb"])
        c = _linear_bn_relu(xc, P[nm + "_l2_W"], P[nm + "_l2_b"],
                            P[nm + "_l2_g"], P[nm + "_l2_bb"])
        idx3, wt3 = _knn(ptf, ptc, 3, True)
        R = B * nf
        idxT = jnp.transpose(idx3, (2, 0, 1)).reshape(3 * R)
        g = _sc_gather(c, idxT)
        wtT = jnp.transpose(wt3, (2, 0, 1)).reshape(3, R)
        return _interp_combine(a, g[:R], g[R:2 * R], g[2 * R:],
                               wtT[0][:, None], wtT[1][:, None],
                               wtT[2][:, None])

    x4d = up(x4, p4t, x5d, p5t, "dec4")
    x3d = up(x3, p3t, x4d, p4t, "dec3")
    x2d = up(x2, p2t, x3d, p3t, "dec2")
    x1d = up(x1, p1t, x2d, p2t, "dec1")

    seg = _head(x1d, P["cls_W1"], P["cls_b1"], P["cls_g"], P["cls_bb"],
                P["cls_W2"], P["cls_b2"])
    edge = _head(x1d, P["edge_W1"], P["edge_b1"], P["edge_g"], P["edge_bb"],
                 P["edge_W2"], P["edge_b2"])
    seg = jnp.transpose(seg.reshape(B, N, -1), (0, 2, 1))
    edge = jnp.transpose(edge.reshape(B, N, -1), (0, 2, 1))
    return seg, edge


# speedup vs baseline: 13.4169x; 2.4674x over previous
import functools

import jax
import jax.numpy as jnp
from jax import lax
from jax.experimental import pallas as pl
from jax.experimental.pallas import tpu as pltpu
from jax.experimental.pallas import tpu_sc as plsc

_EPS = 1e-5


def _dot(a, b):
    return jnp.dot(a.astype(jnp.bfloat16), b.astype(jnp.bfloat16),
                   preferred_element_type=jnp.float32)


def _dotg(a, b, dn):
    return lax.dot_general(a.astype(jnp.bfloat16), b.astype(jnp.bfloat16), dn,
                           preferred_element_type=jnp.float32)


def _sc_gather(table, idx):
    V, C = table.shape
    if C % 128 != 0:
        Cp = ((C + 127) // 128) * 128
        table = jnp.concatenate(
            [table, jnp.zeros((V, Cp - C), jnp.float32)], axis=1)
        return _sc_gather(table, idx)[:, :C]
    (M,) = idx.shape
    NW = 32
    assert M % NW == 0, M
    bpw = M // NW
    nch = max(1, bpw // 128)
    ch = bpw // nch
    assert ch * nch == bpw and ch % 8 == 0 and ch <= 128, (bpw, nch, ch)
    mesh = plsc.VectorSubcoreMesh(core_axis_name="c", subcore_axis_name="s")

    @functools.partial(
        pl.kernel,
        mesh=mesh,
        out_type=jax.ShapeDtypeStruct((M, C), jnp.float32),
        scratch_types=[
            pltpu.VMEM((bpw,), jnp.int32),
            pltpu.VMEM((ch, C), jnp.float32),
            pltpu.SemaphoreType.DMA,
        ],
    )
    def k(table_hbm, idx_hbm, out_hbm, idx_v, rows_v, sem):
        wid = lax.axis_index("s") * 2 + lax.axis_index("c")
        base = wid * bpw
        pltpu.sync_copy(idx_hbm.at[pl.ds(base, bpw)], idx_v)
        for c0 in range(nch):
            pltpu.async_copy(
                table_hbm.at[idx_v.at[pl.ds(c0 * ch, ch)]], rows_v, sem
            ).wait()
            pltpu.sync_copy(rows_v, out_hbm.at[pl.ds(base + c0 * ch, ch)])

    return k(table, idx)


def _bn_tail(y, nrows, g, b, relu):
    mu = jnp.sum(y, axis=0, keepdims=True) * (1.0 / nrows)
    d = y - mu
    var = jnp.sum(d * d, axis=0, keepdims=True) * (1.0 / nrows)
    out = d / jnp.sqrt(var + _EPS) * g + b
    if relu:
        out = jnp.maximum(out, 0.0)
    return out


def _mm_bn_relu_kernel(x_ref, w_ref, bias_ref, g_ref, b_ref, o_ref, *, nrows, relu):
    y = _dot(x_ref[...], w_ref[...])
    y = y + bias_ref[...]
    o_ref[...] = _bn_tail(y, nrows, g_ref[...], b_ref[...], relu)


def _linear_bn_relu(x, W, bias, g, b, relu=True):
    R, _ = x.shape
    C = W.shape[1]
    return pl.pallas_call(
        functools.partial(_mm_bn_relu_kernel, nrows=R, relu=relu),
        out_shape=jax.ShapeDtypeStruct((R, C), jnp.float32),
    )(x, W, bias.reshape(1, C), g.reshape(1, C), b.reshape(1, C))


def _head_kernel(x_ref, w1_ref, b1_ref, g_ref, bb_ref, w2_ref, b2_ref, o_ref, *, nrows):
    y = _dot(x_ref[...], w1_ref[...])
    y = y + b1_ref[...]
    h = _bn_tail(y, nrows, g_ref[...], bb_ref[...], True)
    o_ref[...] = _dot(h, w2_ref[...]) + b2_ref[...]


def _head(x, W1, b1, g, bb, W2, b2):
    R = x.shape[0]
    C2 = W2.shape[1]
    return pl.pallas_call(
        functools.partial(_head_kernel, nrows=R),
        out_shape=jax.ShapeDtypeStruct((R, C2), jnp.float32),
    )(x, W1, b1.reshape(1, -1), g.reshape(1, -1), bb.reshape(1, -1), W2,
      b2.reshape(1, C2))


def _pe_kernel(pix_ref, emb_ref, o_ref, *, nrows):
    oh = (pix_ref[...] == lax.broadcasted_iota(jnp.int32, (nrows, 32), 1)
          ).astype(jnp.float32)
    o_ref[...] = jnp.dot(oh, emb_ref[...], preferred_element_type=jnp.float32,
                         precision=lax.Precision.HIGHEST)


def _pe(pix, emb):
    R = pix.shape[0]
    emb32 = jnp.concatenate(
        [emb, jnp.zeros((32 - emb.shape[0], emb.shape[1]), jnp.float32)], axis=0)
    return pl.pallas_call(
        functools.partial(_pe_kernel, nrows=R),
        out_shape=jax.ShapeDtypeStruct((R, emb.shape[1]), jnp.float32),
    )(pix, emb32)


def _fps_kernel(px_ref, py_ref, pz_ref, sx_ref, sy_ref, sz_ref, *, B, n, m):
    px, py, pz = px_ref[...], py_ref[...], pz_ref[...]
    lane_n = lax.broadcasted_iota(jnp.int32, (B, n), 1)
    lane_m = lax.broadcasted_iota(jnp.int32, (B, m), 1)
    cx, cy, cz = px[:, 0:1], py[:, 0:1], pz[:, 0:1]
    zm = jnp.zeros((B, m), jnp.float32)
    sx = jnp.where(lane_m == 0, cx, zm)
    sy = jnp.where(lane_m == 0, cy, zm)
    sz = jnp.where(lane_m == 0, cz, zm)
    dists0 = jnp.full((B, n), 1e10, jnp.float32)

    def body(i, st):
        dists, sx, sy, sz, cx, cy, cz = st
        dx, dy, dz = px - cx, py - cy, pz - cz
        dcur = (dx * dx + dy * dy) + dz * dz
        dists = jnp.minimum(dists, dcur)
        mx = jnp.max(dists, axis=1, keepdims=True)
        nxt = jnp.min(jnp.where(dists == mx, lane_n, n), axis=1, keepdims=True)
        oh = lane_n == nxt
        cx = jnp.sum(jnp.where(oh, px, 0.0), axis=1, keepdims=True)
        cy = jnp.sum(jnp.where(oh, py, 0.0), axis=1, keepdims=True)
        cz = jnp.sum(jnp.where(oh, pz, 0.0), axis=1, keepdims=True)
        upd = lane_m == i
        sx = jnp.where(upd, cx, sx)
        sy = jnp.where(upd, cy, sy)
        sz = jnp.where(upd, cz, sz)
        return dists, sx, sy, sz, cx, cy, cz

    _, sx, sy, sz, _, _, _ = lax.fori_loop(
        1, m, body, (dists0, sx, sy, sz, cx, cy, cz))
    sx_ref[...] = sx
    sy_ref[...] = sy
    sz_ref[...] = sz


def _fps(pt, m):
    B, _, n = pt.shape
    sh = jax.ShapeDtypeStruct((B, m), jnp.float32)
    sx, sy, sz = pl.pallas_call(
        functools.partial(_fps_kernel, B=B, n=n, m=m),
        out_shape=[sh, sh, sh],
    )(pt[:, 0, :], pt[:, 1, :], pt[:, 2, :])
    selt = jnp.concatenate(
        [sx[:, None, :], sy[:, None, :], sz[:, None, :],
         jnp.zeros((B, 5, m), jnp.float32)], axis=1)
    return selt


def _dist_block(qt, pt):
    qd = _dotg(qt, pt, (((0,), (0,)), ((), ())))
    ones = jnp.ones((8, 1), jnp.float32)
    qn = lax.dot_general(qt * qt, ones, (((0,), (0,)), ((), ())),
                         preferred_element_type=jnp.float32,
                         precision=lax.Precision.HIGHEST)
    p2 = pt * pt
    rn = (p2[0:1] + p2[1:2]) + p2[2:3]
    return qn + rn - 2.0 * qd


def _knn_kernel(qt_ref, pt_ref, idx_ref, wt_ref, *, n, k, blk, n_full, with_w):
    b = pl.program_id(0)
    d = _dist_block(qt_ref[0], pt_ref[0])
    lane_n = lax.broadcasted_iota(jnp.int32, (blk, n), 1)
    col_k = lax.broadcasted_iota(jnp.int32, (blk, k), 1)
    idxs = jnp.zeros((blk, k), jnp.int32)
    vals = jnp.zeros((blk, k), jnp.float32)
    for j in range(k):
        mn = jnp.min(d, axis=1, keepdims=True)
        sel = jnp.min(jnp.where(d == mn, lane_n, n), axis=1, keepdims=True)
        d = jnp.where(lane_n == sel, 1e30, d)
        idxs = jnp.where(col_k == j, sel, idxs)
        if with_w:
            vals = jnp.where(col_k == j, mn, vals)
    idx_ref[0] = idxs + b * n_full
    if with_w:
        sq = jnp.maximum(vals, 0.0)
        w = 1.0 / (sq + 1e-8)
        wt_ref[0] = w / jnp.sum(w, axis=1, keepdims=True)


def _knn(qt, pt, k, with_w):
    B, _, m = qt.shape
    n = pt.shape[2]
    blk = min(m, 256)
    grid = (B, m // blk)
    out_shapes = [jax.ShapeDtypeStruct((B, m, k), jnp.int32),
                  jax.ShapeDtypeStruct((B, m, k), jnp.float32)]
    idx, wt = pl.pallas_call(
        functools.partial(_knn_kernel, n=n, k=k, blk=blk, n_full=n, with_w=with_w),
        grid=grid,
        in_specs=[pl.BlockSpec((1, 8, blk), lambda b, j: (b, 0, j)),
                  pl.BlockSpec((1, 8, n), lambda b, j: (b, 0, 0))],
        out_specs=[pl.BlockSpec((1, blk, k), lambda b, j: (b, j, 0)),
                   pl.BlockSpec((1, blk, k), lambda b, j: (b, j, 0))],
        out_shape=out_shapes,
    )(qt, pt)
    return idx, wt


def _td_y(g_ref, npx_ref, w_ref, blk, k, C8):
    d = g_ref[...] - npx_ref[...][:, None, :]
    return _dot(d.reshape(blk * k, C8), w_ref[...])


def _td_stats_kernel(g_ref, npx_ref, w_ref, s1_ref, s2_ref, *, blk, k, C8, C):
    y = _td_y(g_ref, npx_ref, w_ref, blk, k, C8)
    s1_ref[...] = jnp.sum(y, axis=0).reshape(1, 1, C)
    s2_ref[...] = jnp.sum(y * y, axis=0).reshape(1, 1, C)


def _td_final_kernel(s1_ref, s2_ref, g_ref, npx_ref, w_ref, gam_ref, bet_ref,
                     o_ref, *, nblk, blk, k, C8, C, ntot):
    s1 = jnp.sum(s1_ref[...].reshape(nblk, C), axis=0, keepdims=True)
    s2 = jnp.sum(s2_ref[...].reshape(nblk, C), axis=0, keepdims=True)
    mu = s1 * (1.0 / ntot)
    var = s2 * (1.0 / ntot) - mu * mu
    scale = gam_ref[...] / jnp.sqrt(var + _EPS)
    y = _td_y(g_ref, npx_ref, w_ref, blk, k, C8)
    yn = (y - mu) * scale + bet_ref[...]
    o_ref[...] = jnp.max(jnp.maximum(yn, 0.0).reshape(blk, k, C), axis=1)


def _td_bn_pool(g3, npx, Wf, gam, bet):
    R, k, C8 = g3.shape
    C = Wf.shape[1]
    blk = min(R, 256)
    nblk = R // blk
    wspec = pl.BlockSpec((C8, C), lambda i: (0, 0))
    s1, s2 = pl.pallas_call(
        functools.partial(_td_stats_kernel, blk=blk, k=k, C8=C8, C=C),
        grid=(nblk,),
        in_specs=[pl.BlockSpec((blk, k, C8), lambda i: (i, 0, 0)),
                  pl.BlockSpec((blk, C8), lambda i: (i, 0)),
                  wspec],
        out_specs=[pl.BlockSpec((1, 1, C), lambda i: (i, 0, 0)),
                   pl.BlockSpec((1, 1, C), lambda i: (i, 0, 0))],
        out_shape=[jax.ShapeDtypeStruct((nblk, 1, C), jnp.float32),
                   jax.ShapeDtypeStruct((nblk, 1, C), jnp.float32)],
    )(g3, npx, Wf)
    out = pl.pallas_call(
        functools.partial(_td_final_kernel, nblk=nblk, blk=blk, k=k, C8=C8,
                          C=C, ntot=R * k),
        grid=(nblk,),
        in_specs=[pl.BlockSpec((nblk, 1, C), lambda i: (0, 0, 0)),
                  pl.BlockSpec((nblk, 1, C), lambda i: (0, 0, 0)),
                  pl.BlockSpec((blk, k, C8), lambda i: (i, 0, 0)),
                  pl.BlockSpec((blk, C8), lambda i: (i, 0)),
                  wspec,
                  pl.BlockSpec((1, C), lambda i: (0, 0)),
                  pl.BlockSpec((1, C), lambda i: (0, 0))],
        out_specs=pl.BlockSpec((blk, C), lambda i: (i, 0)),
        out_shape=jax.ShapeDtypeStruct((R, C), jnp.float32),
    )(s1, s2, g3, npx, Wf, gam.reshape(1, C), bet.reshape(1, C))
    return out


def _dec5_kernel(x5_ref, w2_ref, b2_ref, w1a_ref, w1b_ref, b1_ref, g_ref,
                 bb_ref, o_ref, *, B, npt, C):
    x5 = x5_ref[...]
    avg = jnp.sum(x5, axis=1) * (1.0 / npt)
    gf = jnp.maximum(_dot(avg, w2_ref[...]) + b2_ref[...], 0.0)
    y = (_dot(x5.reshape(B * npt, C), w1a_ref[...]).reshape(B, npt, C)
         + _dot(gf, w1b_ref[...])[:, None, :]
         + b1_ref[...][None])
    o_ref[...] = _bn_tail(y.reshape(B * npt, C), B * npt, g_ref[...],
                          bb_ref[...], True)


def _dec5(x5_flat, B, npt, P):
    C = x5_flat.shape[1]
    W1 = P["dec5_l1_W"]
    return pl.pallas_call(
        functools.partial(_dec5_kernel, B=B, npt=npt, C=C),
        out_shape=jax.ShapeDtypeStruct((B * npt, C), jnp.float32),
    )(x5_flat.reshape(B, npt, C), P["dec5_l2_W"],
      P["dec5_l2_b"].reshape(1, C), W1[:C], W1[C:],
      P["dec5_l1_b"].reshape(1, C), P["dec5_l1_g"].reshape(1, C),
      P["dec5_l1_bb"].reshape(1, C))


def _interp_combine_kernel(a_ref, g0_ref, g1_ref, g2_ref, w0_ref, w1_ref,
                           w2_ref, o_ref):
    o_ref[...] = (a_ref[...]
                  + g0_ref[...] * w0_ref[...]
                  + g1_ref[...] * w1_ref[...]
                  + g2_ref[...] * w2_ref[...])


def _interp_combine(a, g0, g1, g2, w0, w1, w2):
    R, C = a.shape
    blk = min(R, 2048)
    rb = pl.BlockSpec((blk, C), lambda i: (i, 0))
    wb = pl.BlockSpec((blk, 1), lambda i: (i, 0))
    return pl.pallas_call(
        _interp_combine_kernel,
        grid=(R // blk,),
        in_specs=[rb, rb, rb, rb, wb, wb, wb],
        out_specs=rb,
        out_shape=jax.ShapeDtypeStruct((R, C), jnp.float32),
    )(a, g0, g1, g2, w0, w1, w2)


def _pad_pt(sel):
    return sel


def kernel(data, point_to_pixel_feat, emb, params):
    P = params
    B, _, N = data.shape
    f32 = jnp.float32

    dataT = jnp.transpose(data, (0, 2, 1)).reshape(B * N, 6)
    pix = jnp.transpose(point_to_pixel_feat, (0, 2, 1))[:, :, 0]
    pix = pix.reshape(B * N, 1).astype(jnp.int32)

    pe = _pe(pix, emb)
    x10 = jnp.concatenate([dataT, pe], axis=1)
    x1 = _linear_bn_relu(x10, P["enc1_W"],
                         jnp.zeros((P["enc1_W"].shape[1],), f32),
                         P["enc1_g"], P["enc1_b"])

    p1t = jnp.concatenate(
        [data[:, :3, :], jnp.zeros((B, 5, N), f32)], axis=1)

    def td(pt, x_flat, lvl):
        n = pt.shape[2]
        m = n // 4
        W = P["enc%d_W" % lvl]
        Ci = W.shape[0] - 3
        Co = W.shape[1]
        C8 = Ci + 8
        Wf = jnp.concatenate([W[:3], jnp.zeros((5, Co), f32), W[3:]], axis=0)
        selt = _fps(pt, m)
        nbr, _ = _knn(selt, pt, 16, False)
        tbl = jnp.concatenate(
            [jnp.transpose(pt, (0, 2, 1)),
             x_flat.reshape(B, n, Ci)], axis=2).reshape(B * n, C8)
        npx = jnp.concatenate(
            [jnp.transpose(selt, (0, 2, 1)),
             jnp.zeros((B, m, Ci), f32)], axis=2).reshape(B * m, C8)
        g = _sc_gather(tbl, nbr.reshape(-1))
        x_out = _td_bn_pool(g.reshape(B * m, 16, C8), npx, Wf,
                            P["enc%d_g" % lvl], P["enc%d_b" % lvl])
        return selt, x_out

    p2t, x2 = td(p1t, x1, 2)
    p3t, x3 = td(p2t, x2, 3)
    p4t, x4 = td(p3t, x3, 4)
    p5t, x5 = td(p4t, x4, 5)

    x5d = _dec5(x5, B, p5t.shape[2], P)

    def up(xf, ptf, xc, ptc, nm):
        nf, nc = ptf.shape[2], ptc.shape[2]
        a = _linear_bn_relu(xf, P[nm + "_l1_W"], P[nm + "_l1_b"],
                            P[nm + "_l1_g"], P[nm + "_l1_bb"])
        c = _linear_bn_relu(xc, P[nm + "_l2_W"], P[nm + "_l2_b"],
                            P[nm + "_l2_g"], P[nm + "_l2_bb"])
        idx3, wt3 = _knn(ptf, ptc, 3, True)
        R = B * nf
        idxT = jnp.transpose(idx3, (2, 0, 1)).reshape(3 * R)
        g = _sc_gather(c, idxT)
        wtT = jnp.transpose(wt3, (2, 0, 1)).reshape(3, R)
        return _interp_combine(a, g[:R], g[R:2 * R], g[2 * R:],
                               wtT[0][:, None], wtT[1][:, None],
                               wtT[2][:, None])

    x4d = up(x4, p4t, x5d, p5t, "dec4")
    x3d = up(x3, p3t, x4d, p4t, "dec3")
    x2d = up(x2, p2t, x3d, p3t, "dec2")
    x1d = up(x1, p1t, x2d, p2t, "dec1")

    seg = _head(x1d, P["cls_W1"], P["cls_b1"], P["cls_g"], P["cls_bb"],
                P["cls_W2"], P["cls_b2"])
    edge = _head(x1d, P["edge_W1"], P["edge_b1"], P["edge_g"], P["edge_bb"],
                 P["edge_W2"], P["edge_b2"])
    seg = jnp.transpose(seg.reshape(B, N, -1), (0, 2, 1))
    edge = jnp.transpose(edge.reshape(B, N, -1), (0, 2, 1))
    return seg, edge
